# baseline (speedup 1.0000x reference)
import functools

import jax
import jax.numpy as jnp
from jax.experimental import pallas as pl
from jax.experimental.pallas import tpu as pltpu


def _gcn_body(nchunks, nbuf, x_ref, w_ref, b_ref, adj_hbm, out_hbm,
              support_ref, bufs_ref, outbuf_ref, in_sems, out_sem):
    block_m = bufs_ref.shape[1]

    def in_copy(i, slot):
        return pltpu.make_async_copy(
            adj_hbm.at[pl.ds(i * block_m, block_m), :],
            bufs_ref.at[slot],
            in_sems.at[slot],
        )

    for j in range(nbuf):
        in_copy(j, j).start()

    support_ref[...] = jnp.dot(
        x_ref[...], w_ref[...], preferred_element_type=jnp.float32
    )

    def step(g, carry):
        for slot in range(nbuf):
            i = g * nbuf + slot
            in_copy(i, slot).wait()
            outbuf_ref[pl.ds(i * block_m, block_m), :] = (
                jnp.dot(
                    bufs_ref[slot], support_ref[...],
                    preferred_element_type=jnp.float32,
                )
                + b_ref[...]
            )

            @pl.when(i + nbuf < nchunks)
            def _():
                in_copy(i + nbuf, slot).start()

        return carry

    jax.lax.fori_loop(0, nchunks // nbuf, step, 0, unroll=False)

    out_copy = pltpu.make_async_copy(outbuf_ref, out_hbm, out_sem)
    out_copy.start()
    out_copy.wait()


@functools.partial(jax.jit, static_argnames=("block_m", "nbuf"))
def _gcn(input, adj, weight, bias, block_m=200, nbuf=5):
    n, in_f = input.shape
    out_f = weight.shape[1]
    nchunks = n // block_m
    return pl.pallas_call(
        functools.partial(_gcn_body, nchunks, nbuf),
        in_specs=[
            pl.BlockSpec(memory_space=pltpu.MemorySpace.VMEM),
            pl.BlockSpec(memory_space=pltpu.MemorySpace.VMEM),
            pl.BlockSpec(memory_space=pltpu.MemorySpace.VMEM),
            pl.BlockSpec(memory_space=pltpu.MemorySpace.HBM),
        ],
        out_specs=pl.BlockSpec(memory_space=pltpu.MemorySpace.HBM),
        out_shape=jax.ShapeDtypeStruct((n, out_f), jnp.float32),
        scratch_shapes=[
            pltpu.VMEM((n, out_f), jnp.float32),
            pltpu.VMEM((nbuf, block_m, n), jnp.float32),
            pltpu.VMEM((n, out_f), jnp.float32),
            pltpu.SemaphoreType.DMA((nbuf,)),
            pltpu.SemaphoreType.DMA,
        ],
    )(input, weight, bias.reshape(1, out_f), adj)


def kernel(input, adj, weight, bias):
    return _gcn(input, adj, weight, bias)


# speedup vs baseline: 1.0326x; 1.0326x over previous
import functools

import jax
import jax.numpy as jnp
from jax.experimental import pallas as pl
from jax.experimental.pallas import tpu as pltpu


def _gcn_body(adj_ref, x_ref, w_ref, b_ref, out_ref, support_ref):
    @pl.when(pl.program_id(0) == 0)
    def _():
        support_ref[...] = jnp.dot(
            x_ref[...], w_ref[...], preferred_element_type=jnp.float32
        )

    out_ref[...] = (
        jnp.dot(adj_ref[...], support_ref[...], preferred_element_type=jnp.float32)
        + b_ref[...]
    )


@functools.partial(jax.jit, static_argnames=("block_m",))
def _gcn(input, adj, weight, bias, block_m=400):
    n, in_f = input.shape
    out_f = weight.shape[1]
    grid = (n // block_m,)
    return pl.pallas_call(
        _gcn_body,
        grid=grid,
        in_specs=[
            pl.BlockSpec((block_m, n), lambda m: (m, 0)),
            pl.BlockSpec((n, in_f), lambda m: (0, 0)),
            pl.BlockSpec((in_f, out_f), lambda m: (0, 0)),
            pl.BlockSpec((1, out_f), lambda m: (0, 0)),
        ],
        out_specs=pl.BlockSpec((block_m, out_f), lambda m: (m, 0)),
        out_shape=jax.ShapeDtypeStruct((n, out_f), jnp.float32),
        scratch_shapes=[pltpu.VMEM((n, out_f), jnp.float32)],
        compiler_params=pltpu.CompilerParams(
            dimension_semantics=("arbitrary",),
        ),
    )(adj, input, weight, bias.reshape(1, out_f))


def kernel(input, adj, weight, bias):
    return _gcn(input, adj, weight, bias)
